# Initial kernel scaffold; baseline (speedup 1.0000x reference)
import jax
import jax.numpy as jnp
from jax.experimental import pallas as pl


def kernel(feat0, feat1, feat2, feat3, inner_w0, inner_b0, layer_w0, layer_b0, inner_w1, inner_b1, layer_w1, layer_b1, inner_w2, inner_b2, layer_w2, layer_b2, inner_w3, inner_b3, layer_w3, layer_b3):
    raise NotImplementedError("write your pallas kernel here")


# speedup vs baseline: 1.0981x; 1.0981x over previous
import jax
import jax.numpy as jnp
from jax.experimental import pallas as pl
from jax.experimental.pallas import tpu as pltpu


def _k1_body(x_ref, w_ref, b_ref, o_ref):
    x = x_ref[0].astype(jnp.bfloat16)
    y = jax.lax.dot_general(x, w_ref[...], (((0,), (0,)), ((), ())),
                            preferred_element_type=jnp.float32)
    o_ref[0] = (y + b_ref[...]).astype(jnp.bfloat16)


def _make_k1_add_body(W):
    def _body(x_ref, w_ref, b_ref, s_ref, o_ref):
        x = x_ref[0].astype(jnp.bfloat16)
        y = jax.lax.dot_general(x, w_ref[...], (((0,), (0,)), ((), ())),
                                preferred_element_type=jnp.float32)
        TS, C = y.shape
        src = s_ref[0].astype(jnp.float32)
        up = jnp.repeat(src, 2, axis=0)
        up = jnp.repeat(up.reshape(TS // (2 * W), W, C), 2, axis=0)
        o_ref[0] = (y + b_ref[...] + up.reshape(TS, C)).astype(jnp.bfloat16)
    return _body


def _k1_tile(H, W, has_add, target=1024):
    if not has_add:
        return min(H * W, target)
    k = 1
    while 4 * k * W <= target and H % (4 * k) == 0:
        k *= 2
    return 2 * k * W


def _lateral(feat, w_oihw, bias, src_flat, src_hw):
    N, Cin, H, W = feat.shape
    C = w_oihw.shape[0]
    HW = H * W
    x = feat.reshape(N, Cin, HW)
    w2 = jnp.transpose(w_oihw[:, :, 0, 0], (1, 0)).astype(jnp.bfloat16)
    b2 = bias.reshape(1, C).astype(jnp.float32)

    TS = _k1_tile(H, W, src_flat is not None)
    grid = (N, HW // TS)
    in_specs = [
        pl.BlockSpec((1, Cin, TS), lambda n, j: (n, 0, j)),
        pl.BlockSpec((Cin, C), lambda n, j: (0, 0)),
        pl.BlockSpec((1, C), lambda n, j: (0, 0)),
    ]
    args = [x, w2, b2]
    if src_flat is None:
        body = _k1_body
    else:
        body = _make_k1_add_body(W)
        in_specs.append(pl.BlockSpec((1, TS // 4, C), lambda n, j: (n, j, 0)))
        args.append(src_flat)

    bytes_acc = (N * HW * Cin * 4 + Cin * C * 2 + C * 4 + N * HW * C * 2
                 + (0 if src_flat is None else N * HW // 4 * C * 2))
    out = pl.pallas_call(
        body,
        out_shape=jax.ShapeDtypeStruct((N, HW, C), jnp.bfloat16),
        grid=grid,
        in_specs=in_specs,
        out_specs=pl.BlockSpec((1, TS, C), lambda n, j: (n, j, 0)),
        compiler_params=pltpu.CompilerParams(
            dimension_semantics=("parallel", "parallel"),
            vmem_limit_bytes=64 * 1024 * 1024,
        ),
        cost_estimate=pl.CostEstimate(
            flops=int(2 * N * HW * Cin * C), transcendentals=0,
            bytes_accessed=int(bytes_acc)),
    )(*args)
    return out


def _make_k2_body(TH, W):
    def _body(x0_ref, x1_ref, w_ref, b_ref, o_ref):
        xw = jnp.concatenate([x0_ref[0], x1_ref[0, :2]], axis=0)
        xcat = jnp.concatenate([xw[:, dx:dx + W, :] for dx in range(3)],
                               axis=-1)
        b = b_ref[...]
        for t in range(TH):
            acc = jnp.dot(xcat[t], w_ref[0],
                          preferred_element_type=jnp.float32)
            acc += jnp.dot(xcat[t + 1], w_ref[1],
                           preferred_element_type=jnp.float32)
            acc += jnp.dot(xcat[t + 2], w_ref[2],
                           preferred_element_type=jnp.float32)
            o_ref[0, t] = acc + b
    return _body


def _k2_row_tile(H):
    d = 1
    for th in range(1, min(16, H) + 1):
        if H % th == 0:
            d = th
    if d == H and d % 2 == 0 and H > 2:
        d //= 2
    return d


def _smooth(inner_flat, w_oihw, bias, N, H, W):
    C = w_oihw.shape[0]
    x = inner_flat.reshape(N, H, W, C)
    TH = _k2_row_tile(H)
    Ht = H // TH
    Hp = H + TH
    xp = jnp.pad(x, ((0, 0), (1, Hp - H - 1), (1, 1), (0, 0)))
    w3 = jnp.transpose(w_oihw, (2, 3, 1, 0)).reshape(3, 3 * C, C)
    w3 = w3.astype(jnp.bfloat16)
    b2 = bias.reshape(1, C).astype(jnp.float32)

    def _xmap(k):
        return lambda n, i: (n, i + k, 0, 0)

    in_specs = [pl.BlockSpec((1, TH, W + 2, C), _xmap(k)) for k in range(2)]
    in_specs += [
        pl.BlockSpec((3, 3 * C, C), lambda n, i: (0, 0, 0)),
        pl.BlockSpec((1, C), lambda n, i: (0, 0)),
    ]
    flops = 2 * N * H * W * 9 * C * C
    bytes_acc = (2 * N * H * (W + 2) * C * 2 + 9 * C * C * 2 + C * 4
                 + N * H * W * C * 4)
    out = pl.pallas_call(
        _make_k2_body(TH, W),
        out_shape=jax.ShapeDtypeStruct((N, H, W, C), jnp.float32),
        grid=(N, Ht),
        in_specs=in_specs,
        out_specs=pl.BlockSpec((1, TH, W, C), lambda n, i: (n, i, 0, 0)),
        compiler_params=pltpu.CompilerParams(
            dimension_semantics=("parallel", "parallel"),
            vmem_limit_bytes=64 * 1024 * 1024,
        ),
        cost_estimate=pl.CostEstimate(
            flops=int(flops), transcendentals=0,
            bytes_accessed=int(bytes_acc)),
    )(xp, xp, w3, b2)
    return out


def kernel(feat0, feat1, feat2, feat3,
           inner_w0, inner_b0, layer_w0, layer_b0,
           inner_w1, inner_b1, layer_w1, layer_b1,
           inner_w2, inner_b2, layer_w2, layer_b2,
           inner_w3, inner_b3, layer_w3, layer_b3):
    feats = [feat0, feat1, feat2, feat3]
    iw = [inner_w0, inner_w1, inner_w2, inner_w3]
    ib = [inner_b0, inner_b1, inner_b2, inner_b3]
    lw = [layer_w0, layer_w1, layer_w2, layer_w3]
    lb = [layer_b0, layer_b1, layer_b2, layer_b3]

    names = ["feat0", "feat1", "feat2", "feat3"]
    results = [None] * 4
    last_inner = None
    last_hw = None
    for idx in range(3, -1, -1):
        N, _, H, W = feats[idx].shape
        last_inner = _lateral(feats[idx], iw[idx], ib[idx],
                              last_inner, last_hw)
        last_hw = (H, W)
        out = _smooth(last_inner, lw[idx], lb[idx], N, H, W)
        results[idx] = jnp.transpose(out, (0, 3, 1, 2))

    from collections import OrderedDict
    return OrderedDict(zip(names, results))
